# baseline (speedup 1.0000x reference)
import functools

import jax
import jax.numpy as jnp
from jax import lax
from jax.experimental import pallas as pl
from jax.experimental.pallas import tpu as pltpu
from jax.experimental.pallas import tpu_sc as plsc

NC = 2
NS = 16
NW = NC * NS
CH = 128
BL = 200


def _rup(n, m):
    return ((n + m - 1) // m) * m


def _sc_gather(table, idx):
    B0 = idx.shape[0]
    Bp = _rup(B0, 8 * NW)
    if Bp != B0:
        idx = jnp.pad(idx, (0, Bp - B0))
    V, d = table.shape
    per_w = Bp // NW
    n_full, tail = divmod(per_w, CH)
    n_pair = n_full // 2
    mesh = plsc.VectorSubcoreMesh(core_axis_name="c", subcore_axis_name="s")

    def body(table_hbm, idx_hbm, out_hbm, idx_v, rows0, rows1, sem):
        wid = lax.axis_index("s") * NC + lax.axis_index("c")
        base = wid * per_w
        pltpu.sync_copy(idx_hbm.at[pl.ds(base, per_w)], idx_v)

        def fire(j, buf):
            pltpu.async_copy(table_hbm.at[idx_v.at[pl.ds(j * CH, CH)]],
                             buf, sem)

        def drain_out(j, buf):
            pltpu.make_async_copy(table_hbm.at[idx_v.at[pl.ds(j * CH, CH)]],
                                  buf, sem).wait()
            pltpu.sync_copy(buf, out_hbm.at[pl.ds(base + j * CH, CH)])

        fire(0, rows0)

        def pair(p, carry):
            a = 2 * p
            fire(a + 1, rows1)
            drain_out(a, rows0)

            @pl.when(a + 2 < n_full)
            def _():
                fire(a + 2, rows0)

            drain_out(a + 1, rows1)
            return carry

        lax.fori_loop(0, n_pair, pair, 0)
        if n_full % 2:
            drain_out(n_full - 1, rows0)
        if tail:
            off = n_full * CH
            pltpu.async_copy(
                table_hbm.at[idx_v.at[pl.ds(off, tail)]],
                rows1.at[pl.ds(0, tail)], sem).wait()
            pltpu.sync_copy(rows1.at[pl.ds(0, tail)],
                            out_hbm.at[pl.ds(base + off, tail)])

    fn = pl.kernel(
        body,
        out_type=jax.ShapeDtypeStruct((Bp, d), jnp.float32),
        mesh=mesh,
        scratch_types=[
            pltpu.VMEM((per_w,), jnp.int32),
            pltpu.VMEM((CH, d), jnp.float32),
            pltpu.VMEM((CH, d), jnp.float32),
            pltpu.SemaphoreType.DMA,
        ],
    )
    out = fn(table, idx)
    return out[:B0] if Bp != B0 else out


def _sc_segsum(rows, idx, n_seg):
    B0 = rows.shape[0]
    CHI = 2048
    GB = 32
    Bp = _rup(B0, CHI)
    if Bp != B0:
        idx = jnp.pad(idx, (0, Bp - B0), constant_values=-1)
    d = rows.shape[1]
    ds_rows = _rup(_rup(n_seg, NW) // NW, 8)
    n_chunks = Bp // CHI
    zs = jnp.zeros((ds_rows, d), jnp.float32)
    mesh = plsc.VectorSubcoreMesh(core_axis_name="c", subcore_axis_name="s")

    def body(rows_hbm, idx_hbm, zs_hbm, out_hbm,
             idxbuf, ridbuf, locbuf, rowbuf, acc, sem):
        wid = lax.axis_index("s") * NC + lax.axis_index("c")
        lo = wid * ds_rows
        pltpu.sync_copy(zs_hbm, acc)

        def chunk(j, carry):
            off = j * CHI
            pltpu.sync_copy(idx_hbm.at[pl.ds(off, CHI)], idxbuf)
            zero16 = jnp.full((16,), 0, jnp.int32)
            for i in range(CHI // 16):
                ridbuf[pl.ds(i * 16, 16)] = zero16
            cnt = jnp.int32(0)
            lane = lax.iota(jnp.int32, 16)
            for i in range(CHI // 16):
                sl = pl.ds(i * 16, 16)
                v = idxbuf[sl]
                ok = (v >= lo) & (v < lo + ds_rows)
                pc = plsc.cumsum(jnp.where(ok, 1, 0))
                pos = jnp.where(ok, cnt + pc - 1, CHI + 16 + lane)
                plsc.store_scatter(ridbuf, [pos], off + i * 16 + lane)
                plsc.store_scatter(locbuf, [pos], v - lo)
                cnt = cnt + pc[15]

            def blk(b, carry2):
                pltpu.async_copy(
                    rows_hbm.at[ridbuf.at[pl.ds(b * GB, GB)]],
                    rowbuf, sem).wait()

                def add_row(r, carry3):
                    dloc = locbuf[pl.ds(r, 16)][0]
                    rr = r - b * GB
                    for i in range(d // 16):
                        sl2 = pl.ds(i * 16, 16)
                        plsc.addupdate(acc.at[dloc, sl2], rowbuf[rr, sl2])
                    return carry3

                lax.fori_loop(b * GB, jnp.minimum(cnt, (b + 1) * GB),
                              add_row, 0)
                return carry2

            lax.fori_loop(0, (cnt + GB - 1) // GB, blk, 0)
            return carry

        lax.fori_loop(0, n_chunks, chunk, 0)
        pltpu.sync_copy(acc, out_hbm.at[pl.ds(wid * ds_rows, ds_rows)])

    fn = pl.kernel(
        body,
        out_type=jax.ShapeDtypeStruct((NW * ds_rows, d), jnp.float32),
        mesh=mesh,
        compiler_params=pltpu.CompilerParams(needs_layout_passes=False),
        scratch_types=[
            pltpu.VMEM((CHI,), jnp.int32),
            pltpu.VMEM((CHI + 32,), jnp.int32),
            pltpu.VMEM((CHI + 32,), jnp.int32),
            pltpu.VMEM((GB, d), jnp.float32),
            pltpu.VMEM((ds_rows, d), jnp.float32),
            pltpu.SemaphoreType.DMA,
        ],
    )
    out = fn(rows, idx, zs)
    return out[:n_seg]


def _scale_body(x_ref, w_ref, o_ref):
    o_ref[...] = x_ref[...] * w_ref[...]


def _scale_rows(x, w):
    N, d = x.shape
    return pl.pallas_call(
        _scale_body,
        out_shape=jax.ShapeDtypeStruct((N, d), x.dtype),
        grid=(N // BL,),
        in_specs=[pl.BlockSpec((BL, d), lambda i: (i, 0)),
                  pl.BlockSpec((BL, 1), lambda i: (i, 0))],
        out_specs=pl.BlockSpec((BL, d), lambda i: (i, 0)),
    )(x, w)


def _l2n_body(x_ref, o_ref):
    x = x_ref[...]
    n = jnp.sqrt(jnp.sum(x * x, axis=1, keepdims=True))
    o_ref[...] = x / jnp.maximum(n, 1e-12)


def _l2norm(x):
    N, d = x.shape
    return pl.pallas_call(
        _l2n_body,
        out_shape=jax.ShapeDtypeStruct((N, d), x.dtype),
        grid=(N // BL,),
        in_specs=[pl.BlockSpec((BL, d), lambda i: (i, 0))],
        out_specs=pl.BlockSpec((BL, d), lambda i: (i, 0)),
    )(x)


def _colstats_body(x_ref, o_ref):
    i = pl.program_id(0)
    x = x_ref[...]
    s = jnp.sum(x, axis=0, keepdims=True)
    q = jnp.sum(x * x, axis=0, keepdims=True)
    blk = jnp.concatenate([s, q, jnp.zeros((6, x.shape[1]), x.dtype)], axis=0)

    @pl.when(i == 0)
    def _():
        o_ref[...] = blk

    @pl.when(i != 0)
    def _():
        o_ref[...] = o_ref[...] + blk


def _colstats(x):
    N, d = x.shape
    return pl.pallas_call(
        _colstats_body,
        out_shape=jax.ShapeDtypeStruct((8, d), x.dtype),
        grid=(N // BL,),
        in_specs=[pl.BlockSpec((BL, d), lambda i: (i, 0))],
        out_specs=pl.BlockSpec((8, d), lambda i: (0, 0)),
    )(x)


def _mlp_body(xa_ref, xb_ref, xc_ref, res_ref, sa_ref, sb_ref, sc_ref,
              ga_ref, gb_ref, gc_ref, w1_ref, b1_ref, w2_ref, b2_ref, o_ref,
              *, n_rows, d):
    h = None
    pieces = ((xa_ref, sa_ref, ga_ref, 0), (xb_ref, sb_ref, gb_ref, 1),
              (xc_ref, sc_ref, gc_ref, 2))
    for x_ref, s_ref, g_ref, k in pieces:
        mean = s_ref[0:1, :] / n_rows
        var = s_ref[1:2, :] / n_rows - mean * mean
        scale = g_ref[0:1, :] * lax.rsqrt(var + 1e-5)
        shift = g_ref[1:2, :] - mean * scale
        xh = x_ref[...] * scale + shift
        part = jnp.dot(xh.astype(jnp.bfloat16),
                       w1_ref[k * d:(k + 1) * d, :].astype(jnp.bfloat16),
                       preferred_element_type=jnp.float32)
        h = part if h is None else h + part
    h = h + b1_ref[0:1, :]
    h = 0.5 * h * (1.0 + lax.erf(h * 0.7071067811865476))
    y = jnp.dot(h.astype(jnp.bfloat16), w2_ref[...].astype(jnp.bfloat16),
                preferred_element_type=jnp.float32)
    o_ref[...] = y + b2_ref[0:1, :] + res_ref[...]


def _update(xa, xb, xc, resid, gamma, beta, w1, b1, w2, b2):
    N, d = xa.shape
    dh = w1.shape[1]
    sa, sb, sc = _colstats(xa), _colstats(xb), _colstats(xc)
    g3 = gamma.reshape(3, d)
    b3 = beta.reshape(3, d)
    z6 = jnp.zeros((6, d), jnp.float32)
    gbs = [jnp.concatenate([g3[k:k + 1], b3[k:k + 1], z6], axis=0)
           for k in range(3)]
    b1p = jnp.concatenate([b1.reshape(1, dh), jnp.zeros((7, dh), jnp.float32)], axis=0)
    b2p = jnp.concatenate([b2.reshape(1, d), jnp.zeros((7, d), jnp.float32)], axis=0)
    row_spec = pl.BlockSpec((BL, d), lambda i: (i, 0))
    fix8 = pl.BlockSpec((8, d), lambda i: (0, 0))
    return pl.pallas_call(
        functools.partial(_mlp_body, n_rows=float(N), d=d),
        out_shape=jax.ShapeDtypeStruct((N, d), jnp.float32),
        grid=(N // BL,),
        in_specs=[
            row_spec, row_spec, row_spec, row_spec,
            fix8, fix8, fix8, fix8, fix8, fix8,
            pl.BlockSpec((3 * d, dh), lambda i: (0, 0)),
            pl.BlockSpec((8, dh), lambda i: (0, 0)),
            pl.BlockSpec((dh, d), lambda i: (0, 0)),
            fix8,
        ],
        out_specs=row_spec,
    )(xa, xb, xc, resid, sa, sb, sc, gbs[0], gbs[1], gbs[2], w1, b1p, w2, b2p)


def kernel(nodes, edges, snodes, sedges, graph, bgraph, bweights, sgraph, sweights,
           node_W1, node_b1, node_W2, node_b2, node_gamma, node_beta,
           edge_W1, edge_b1, edge_W2, edge_b2, edge_gamma, edge_beta,
           snode_W1, snode_b1, snode_W2, snode_b2, snode_gamma, snode_beta,
           sedge_W1, sedge_b1, sedge_W2, sedge_b2, sedge_gamma, sedge_beta):
    g0, g1 = graph[0], graph[1]
    bg0, bg1 = bgraph[0], bgraph[1]
    sg0, sg1 = sgraph[0], sgraph[1]
    n_nodes = nodes.shape[0]
    n_snodes = snodes.shape[0]

    ln_nodes = _l2norm(nodes)
    gb = _sc_gather(ln_nodes, bg0)
    node_msgs = _sc_segsum(_scale_rows(gb, bweights), bg1, n_snodes)
    ln_sedges = _l2norm(sedges)
    sedge_msgs = _sc_segsum(_scale_rows(ln_sedges, sweights), sg1, n_snodes)
    snodes2 = _update(snodes, sedge_msgs, node_msgs, snodes,
                      snode_gamma, snode_beta, snode_W1, snode_b1, snode_W2, snode_b2)

    ln_s2 = _l2norm(snodes2)
    gs = _sc_gather(ln_s2, bg1)
    snode_msgs = _sc_segsum(_scale_rows(gs, bweights), bg0, n_nodes)
    edge_msgs = _sc_segsum(edges, g1, n_nodes)
    nodes2 = _update(nodes, edge_msgs, snode_msgs, nodes,
                     node_gamma, node_beta, node_W1, node_b1, node_W2, node_b2)

    n_se = sg0.shape[0]
    s_both = _sc_gather(snodes2, sgraph.reshape(-1))
    sedges2 = _update(s_both[:n_se], s_both[n_se:], sedges, sedges,
                      sedge_gamma, sedge_beta, sedge_W1, sedge_b1, sedge_W2, sedge_b2)

    n_e = g0.shape[0]
    g_both = _sc_gather(nodes2, graph.reshape(-1))
    edges2 = _update(g_both[:n_e], g_both[n_e:], edges, edges,
                     edge_gamma, edge_beta, edge_W1, edge_b1, edge_W2, edge_b2)

    return nodes2, edges2, snodes2, sedges2


# speedup vs baseline: 1.0382x; 1.0382x over previous
import functools

import jax
import jax.numpy as jnp
from jax import lax
from jax.experimental import pallas as pl
from jax.experimental.pallas import tpu as pltpu
from jax.experimental.pallas import tpu_sc as plsc

NC = 2
NS = 16
NW = NC * NS
CH = 128
BL = 200


def _rup(n, m):
    return ((n + m - 1) // m) * m


def _sc_gather(table, idx):
    B0 = idx.shape[0]
    Bp = _rup(B0, 8 * NW)
    if Bp != B0:
        idx = jnp.pad(idx, (0, Bp - B0))
    V, d = table.shape
    per_w = Bp // NW
    n_full, tail = divmod(per_w, CH)
    n_pair = n_full // 2
    mesh = plsc.VectorSubcoreMesh(core_axis_name="c", subcore_axis_name="s")

    def body(table_hbm, idx_hbm, out_hbm, idx_v, rows0, rows1, sem):
        wid = lax.axis_index("s") * NC + lax.axis_index("c")
        base = wid * per_w
        pltpu.sync_copy(idx_hbm.at[pl.ds(base, per_w)], idx_v)

        def fire(j, buf):
            pltpu.async_copy(table_hbm.at[idx_v.at[pl.ds(j * CH, CH)]],
                             buf, sem)

        def drain_out(j, buf):
            pltpu.make_async_copy(table_hbm.at[idx_v.at[pl.ds(j * CH, CH)]],
                                  buf, sem).wait()
            pltpu.sync_copy(buf, out_hbm.at[pl.ds(base + j * CH, CH)])

        fire(0, rows0)

        def pair(p, carry):
            a = 2 * p
            fire(a + 1, rows1)
            drain_out(a, rows0)

            @pl.when(a + 2 < n_full)
            def _():
                fire(a + 2, rows0)

            drain_out(a + 1, rows1)
            return carry

        lax.fori_loop(0, n_pair, pair, 0)
        if n_full % 2:
            drain_out(n_full - 1, rows0)
        if tail:
            off = n_full * CH
            pltpu.async_copy(
                table_hbm.at[idx_v.at[pl.ds(off, tail)]],
                rows1.at[pl.ds(0, tail)], sem).wait()
            pltpu.sync_copy(rows1.at[pl.ds(0, tail)],
                            out_hbm.at[pl.ds(base + off, tail)])

    fn = pl.kernel(
        body,
        out_type=jax.ShapeDtypeStruct((Bp, d), jnp.float32),
        mesh=mesh,
        scratch_types=[
            pltpu.VMEM((per_w,), jnp.int32),
            pltpu.VMEM((CH, d), jnp.float32),
            pltpu.VMEM((CH, d), jnp.float32),
            pltpu.SemaphoreType.DMA,
        ],
    )
    out = fn(table, idx)
    return out[:B0] if Bp != B0 else out


def _sc_segsum(rows, idx, n_seg):
    B0 = rows.shape[0]
    CHI = 2048
    GB = 32
    Bp = _rup(B0, CHI)
    if Bp != B0:
        idx = jnp.pad(idx, (0, Bp - B0), constant_values=-1)
    d = rows.shape[1]
    ds_rows = _rup(_rup(n_seg, NW) // NW, 8)
    n_chunks = Bp // CHI
    zs = jnp.zeros((ds_rows, d), jnp.float32)
    mesh = plsc.VectorSubcoreMesh(core_axis_name="c", subcore_axis_name="s")

    def body(rows_hbm, idx_hbm, zs_hbm, out_hbm,
             idxbuf, ridbuf, locbuf, rowbuf, acc, sem):
        wid = lax.axis_index("s") * NC + lax.axis_index("c")
        lo = wid * ds_rows
        pltpu.sync_copy(zs_hbm, acc)

        def chunk(j, carry):
            off = j * CHI
            pltpu.sync_copy(idx_hbm.at[pl.ds(off, CHI)], idxbuf)
            zero16 = jnp.full((16,), 0, jnp.int32)
            for i in range(CHI // 16):
                ridbuf[pl.ds(i * 16, 16)] = zero16
            cnt = jnp.int32(0)
            lane = lax.iota(jnp.int32, 16)
            for i in range(CHI // 16):
                sl = pl.ds(i * 16, 16)
                v = idxbuf[sl]
                ok = (v >= lo) & (v < lo + ds_rows)
                pc = plsc.cumsum(jnp.where(ok, 1, 0))
                pos = jnp.where(ok, cnt + pc - 1, CHI + 16 + lane)
                plsc.store_scatter(ridbuf, [pos], off + i * 16 + lane)
                plsc.store_scatter(locbuf, [pos], v - lo)
                cnt = cnt + pc[15]

            def blk(b, carry2):
                pltpu.async_copy(
                    rows_hbm.at[ridbuf.at[pl.ds(b * GB, GB)]],
                    rowbuf, sem).wait()

                def add_row(r, carry3):
                    dloc = locbuf[pl.ds(r, 16)][0]
                    rr = r - b * GB
                    for i in range(d // 16):
                        sl2 = pl.ds(i * 16, 16)
                        plsc.addupdate(acc.at[dloc, sl2], rowbuf[rr, sl2])
                    return carry3

                lax.fori_loop(b * GB, jnp.minimum(cnt, (b + 1) * GB),
                              add_row, 0)
                return carry2

            lax.fori_loop(0, (cnt + GB - 1) // GB, blk, 0)
            return carry

        lax.fori_loop(0, n_chunks, chunk, 0)
        pltpu.sync_copy(acc, out_hbm.at[pl.ds(wid * ds_rows, ds_rows)])

    fn = pl.kernel(
        body,
        out_type=jax.ShapeDtypeStruct((NW * ds_rows, d), jnp.float32),
        mesh=mesh,
        compiler_params=pltpu.CompilerParams(needs_layout_passes=False),
        scratch_types=[
            pltpu.VMEM((CHI,), jnp.int32),
            pltpu.VMEM((CHI + 32,), jnp.int32),
            pltpu.VMEM((CHI + 32,), jnp.int32),
            pltpu.VMEM((GB, d), jnp.float32),
            pltpu.VMEM((ds_rows, d), jnp.float32),
            pltpu.SemaphoreType.DMA,
        ],
    )
    out = fn(rows, idx, zs)
    return out[:n_seg]


def _scale_body(x_ref, w_ref, o_ref):
    o_ref[...] = x_ref[...] * w_ref[...]


def _scale_rows(x, w):
    N, d = x.shape
    return pl.pallas_call(
        _scale_body,
        out_shape=jax.ShapeDtypeStruct((N, d), x.dtype),
        grid=(N // BL,),
        in_specs=[pl.BlockSpec((BL, d), lambda i: (i, 0)),
                  pl.BlockSpec((BL, 1), lambda i: (i, 0))],
        out_specs=pl.BlockSpec((BL, d), lambda i: (i, 0)),
    )(x, w)


def _l2n_body(x_ref, o_ref):
    x = x_ref[...]
    n = jnp.sqrt(jnp.sum(x * x, axis=1, keepdims=True))
    o_ref[...] = x / jnp.maximum(n, 1e-12)


def _l2norm(x):
    N, d = x.shape
    return pl.pallas_call(
        _l2n_body,
        out_shape=jax.ShapeDtypeStruct((N, d), x.dtype),
        grid=(N // BL,),
        in_specs=[pl.BlockSpec((BL, d), lambda i: (i, 0))],
        out_specs=pl.BlockSpec((BL, d), lambda i: (i, 0)),
    )(x)


def _colstats_body(x_ref, o_ref):
    i = pl.program_id(0)
    x = x_ref[...]
    s = jnp.sum(x, axis=0, keepdims=True)
    q = jnp.sum(x * x, axis=0, keepdims=True)
    blk = jnp.concatenate([s, q, jnp.zeros((6, x.shape[1]), x.dtype)], axis=0)

    @pl.when(i == 0)
    def _():
        o_ref[...] = blk

    @pl.when(i != 0)
    def _():
        o_ref[...] = o_ref[...] + blk


def _colstats(x):
    N, d = x.shape
    return pl.pallas_call(
        _colstats_body,
        out_shape=jax.ShapeDtypeStruct((8, d), x.dtype),
        grid=(N // BL,),
        in_specs=[pl.BlockSpec((BL, d), lambda i: (i, 0))],
        out_specs=pl.BlockSpec((8, d), lambda i: (0, 0)),
    )(x)


def _mlp_body(xa_ref, xb_ref, xc_ref, res_ref, sa_ref, sb_ref, sc_ref,
              ga_ref, gb_ref, gc_ref, w1_ref, b1_ref, w2_ref, b2_ref, o_ref,
              *, n_rows, d):
    h = None
    pieces = ((xa_ref, sa_ref, ga_ref, 0), (xb_ref, sb_ref, gb_ref, 1),
              (xc_ref, sc_ref, gc_ref, 2))
    for x_ref, s_ref, g_ref, k in pieces:
        mean = s_ref[0:1, :] / n_rows
        var = s_ref[1:2, :] / n_rows - mean * mean
        scale = g_ref[0:1, :] * lax.rsqrt(var + 1e-5)
        shift = g_ref[1:2, :] - mean * scale
        xh = x_ref[...] * scale + shift
        part = jnp.dot(xh.astype(jnp.bfloat16),
                       w1_ref[k * d:(k + 1) * d, :].astype(jnp.bfloat16),
                       preferred_element_type=jnp.float32)
        h = part if h is None else h + part
    h = h + b1_ref[0:1, :]
    h = 0.5 * h * (1.0 + lax.erf(h * 0.7071067811865476))
    y = jnp.dot(h.astype(jnp.bfloat16), w2_ref[...].astype(jnp.bfloat16),
                preferred_element_type=jnp.float32)
    o_ref[...] = y + b2_ref[0:1, :] + res_ref[...]


def _update(xa, xb, xc, resid, gamma, beta, w1, b1, w2, b2):
    N, d = xa.shape
    dh = w1.shape[1]
    sa, sb, sc = _colstats(xa), _colstats(xb), _colstats(xc)
    g3 = gamma.reshape(3, d)
    b3 = beta.reshape(3, d)
    z6 = jnp.zeros((6, d), jnp.float32)
    gbs = [jnp.concatenate([g3[k:k + 1], b3[k:k + 1], z6], axis=0)
           for k in range(3)]
    b1p = jnp.concatenate([b1.reshape(1, dh), jnp.zeros((7, dh), jnp.float32)], axis=0)
    b2p = jnp.concatenate([b2.reshape(1, d), jnp.zeros((7, d), jnp.float32)], axis=0)
    row_spec = pl.BlockSpec((BL, d), lambda i: (i, 0))
    fix8 = pl.BlockSpec((8, d), lambda i: (0, 0))
    return pl.pallas_call(
        functools.partial(_mlp_body, n_rows=float(N), d=d),
        out_shape=jax.ShapeDtypeStruct((N, d), jnp.float32),
        grid=(N // BL,),
        in_specs=[
            row_spec, row_spec, row_spec, row_spec,
            fix8, fix8, fix8, fix8, fix8, fix8,
            pl.BlockSpec((3 * d, dh), lambda i: (0, 0)),
            pl.BlockSpec((8, dh), lambda i: (0, 0)),
            pl.BlockSpec((dh, d), lambda i: (0, 0)),
            fix8,
        ],
        out_specs=row_spec,
    )(xa, xb, xc, resid, sa, sb, sc, gbs[0], gbs[1], gbs[2], w1, b1p, w2, b2p)


def kernel(nodes, edges, snodes, sedges, graph, bgraph, bweights, sgraph, sweights,
           node_W1, node_b1, node_W2, node_b2, node_gamma, node_beta,
           edge_W1, edge_b1, edge_W2, edge_b2, edge_gamma, edge_beta,
           snode_W1, snode_b1, snode_W2, snode_b2, snode_gamma, snode_beta,
           sedge_W1, sedge_b1, sedge_W2, sedge_b2, sedge_gamma, sedge_beta):
    g0, g1 = graph[0], graph[1]
    bg0, bg1 = bgraph[0], bgraph[1]
    sg0, sg1 = sgraph[0], sgraph[1]
    n_nodes = nodes.shape[0]
    n_snodes = snodes.shape[0]

    ln_nodes = _l2norm(nodes)
    gb = _sc_gather(ln_nodes, bg0)
    node_msgs = _sc_segsum(_scale_rows(gb, bweights), bg1, n_snodes)
    ln_sedges = _l2norm(sedges)
    sedge_msgs = _sc_segsum(_scale_rows(ln_sedges, sweights), sg1, n_snodes)
    snodes2 = _update(snodes, sedge_msgs, node_msgs, snodes,
                      snode_gamma, snode_beta, snode_W1, snode_b1, snode_W2, snode_b2)

    ln_s2 = _l2norm(snodes2)
    gs = _sc_gather(ln_s2, bg1)
    snode_msgs = _sc_segsum(_scale_rows(gs, bweights), bg0, n_nodes)
    edge_msgs = _sc_segsum(edges, g1, n_nodes)
    nodes2 = _update(nodes, edge_msgs, snode_msgs, nodes,
                     node_gamma, node_beta, node_W1, node_b1, node_W2, node_b2)

    sa = _sc_gather(snodes2, sg0)
    sb_rows = _sc_gather(snodes2, sg1)
    sedges2 = _update(sa, sb_rows, sedges, sedges,
                      sedge_gamma, sedge_beta, sedge_W1, sedge_b1, sedge_W2, sedge_b2)

    ga_rows = _sc_gather(nodes2, g0)
    gb_rows = _sc_gather(nodes2, g1)
    edges2 = _update(ga_rows, gb_rows, edges, edges,
                     edge_gamma, edge_beta, edge_W1, edge_b1, edge_W2, edge_b2)

    return nodes2, edges2, snodes2, sedges2
